# baseline (speedup 1.0000x reference)
import functools

import jax
import jax.numpy as jnp
from jax import lax
from jax.experimental import pallas as pl
from jax.experimental.pallas import tpu as pltpu
from jax.experimental.pallas import tpu_sc as plsc

ALPHA = 200.0
B, N, M = 16, 512, 512
LANES = 16
CHUNK = 16
NCH = N // CHUNK
CVECS = M // LANES


def _all_lanes_reduce(x, op, lanes):
    for sh in (8, 4, 2, 1):
        x = op(x, x.at[lanes ^ sh].get(mode="promise_in_bounds"))
    return x


def _sm_body(s_hbm, nrow_hbm, ncol_hbm, out_hbm, buf, zbuf, nrow_v, ncol_v):
    wid = lax.axis_index("s") * 2 + lax.axis_index("c")

    pltpu.sync_copy(nrow_hbm, nrow_v)
    pltpu.sync_copy(ncol_hbm, ncol_v)

    lanes = lax.iota(jnp.int32, LANES)

    def _zinit(j, carry):
        r = j // CVECS
        c = j % CVECS
        zbuf[r, pl.ds(c * LANES, LANES)] = jnp.zeros((LANES,), jnp.float32)
        return carry

    lax.fori_loop(0, CHUNK * CVECS, _zinit, 0)

    nv = nrow_v[...]
    mv = ncol_v[...]

    for b in range(B):
        n = nv[b]
        m = mv[b]
        ch = lax.rem(wid + 2 * b, NCH)
        r0 = ch * CHUNK

        @pl.when(r0 < n)
        def _compute():
            pltpu.sync_copy(s_hbm.at[b, pl.ds(r0, CHUNK), :], buf)

            def _row(r, carry):
                def _p1(c, acc):
                    x = buf[r, pl.ds(c * LANES, LANES)]
                    colv = (c * LANES + lanes) < m
                    t = jnp.where(colv, x * ALPHA, -3.0e38)
                    return jnp.maximum(acc, t)

                mvec = lax.fori_loop(
                    0, CVECS, _p1, jnp.full((LANES,), -3.0e38, jnp.float32))
                rowmax = _all_lanes_reduce(mvec, jnp.maximum, lanes)

                def _p2(c, acc):
                    x = buf[r, pl.ds(c * LANES, LANES)]
                    colv = (c * LANES + lanes) < m
                    e = jnp.where(colv, jnp.exp(x * ALPHA - rowmax), 0.0)
                    buf[r, pl.ds(c * LANES, LANES)] = e
                    return acc + e

                svec = lax.fori_loop(
                    0, CVECS, _p2, jnp.zeros((LANES,), jnp.float32))
                denom = _all_lanes_reduce(svec, jnp.add, lanes)

                rf = jnp.where(r0 + r < n, 1.0, 0.0)
                scale = rf / denom

                def _p3(c, carry2):
                    buf[r, pl.ds(c * LANES, LANES)] = (
                        buf[r, pl.ds(c * LANES, LANES)] * scale)
                    return carry2

                lax.fori_loop(0, CVECS, _p3, 0)
                return carry

            lax.fori_loop(0, CHUNK, _row, 0)
            pltpu.sync_copy(buf, out_hbm.at[b, pl.ds(r0, CHUNK), :])

        @pl.when(r0 >= n)
        def _zero():
            pltpu.sync_copy(zbuf, out_hbm.at[b, pl.ds(r0, CHUNK), :])


@jax.jit
def _sm_call(s, nrow_gt, ncol_gt):
    mesh = plsc.VectorSubcoreMesh(core_axis_name="c", subcore_axis_name="s")
    return pl.kernel(
        _sm_body,
        mesh=mesh,
        out_type=jax.ShapeDtypeStruct((B, N, M), jnp.float32),
        scratch_types=[
            pltpu.VMEM((CHUNK, M), jnp.float32),
            pltpu.VMEM((CHUNK, M), jnp.float32),
            pltpu.VMEM((LANES,), jnp.int32),
            pltpu.VMEM((LANES,), jnp.int32),
        ],
    )(s, nrow_gt, ncol_gt)


def kernel(s, nrow_gt, ncol_gt):
    return _sm_call(s, nrow_gt, ncol_gt)


# speedup vs baseline: 1.5398x; 1.5398x over previous
import functools

import jax
import jax.numpy as jnp
from jax import lax
from jax.experimental import pallas as pl
from jax.experimental.pallas import tpu as pltpu
from jax.experimental.pallas import tpu_sc as plsc

ALPHA = 200.0
B, N, M = 16, 512, 512
LANES = 16
CHUNK = 16
NCH = N // CHUNK
CVECS = M // LANES
UNROLL = 4


def _all_lanes_reduce(x, op, lanes):
    for sh in (8, 4, 2, 1):
        x = op(x, x.at[lanes ^ sh].get(mode="promise_in_bounds"))
    return x


def _sm_body(s_hbm, nrow_hbm, ncol_hbm, out_hbm, buf, zbuf, nrow_v, ncol_v):
    wid = lax.axis_index("s") * 2 + lax.axis_index("c")

    pltpu.sync_copy(nrow_hbm, nrow_v)
    pltpu.sync_copy(ncol_hbm, ncol_v)

    lanes = lax.iota(jnp.int32, LANES)

    def _zinit(j, carry):
        r = j // CVECS
        c = j % CVECS
        zbuf[r, pl.ds(c * LANES, LANES)] = jnp.zeros((LANES,), jnp.float32)
        return carry

    lax.fori_loop(0, CHUNK * CVECS, _zinit, 0)

    nv = nrow_v[...]
    mv = ncol_v[...]

    for b in range(B):
        n = nv[b]
        m = mv[b]
        ch = lax.rem(wid + 2 * b, NCH)
        r0 = ch * CHUNK

        nblk = (m + LANES - 1) // LANES
        ngrp = (nblk + UNROLL - 1) // UNROLL
        zvec = jnp.zeros((LANES,), jnp.float32)

        @pl.when(r0 < n)
        def _compute():
            pltpu.sync_copy(s_hbm.at[b, pl.ds(r0, CHUNK), :], buf)
            rv = jnp.clip(n - r0, 0, CHUNK)

            def _row(r, carry):
                def _p1(g, acc):
                    base = g * (UNROLL * LANES)
                    for u in range(UNROLL):
                        x = buf[r, pl.ds(base + u * LANES, LANES)]
                        colv = (base + u * LANES + lanes) < m
                        acc = jnp.maximum(
                            acc, jnp.where(colv, x * ALPHA, -3.0e38))
                    return acc

                mvec = lax.fori_loop(
                    0, ngrp, _p1, jnp.full((LANES,), -3.0e38, jnp.float32))
                rowmax = _all_lanes_reduce(mvec, jnp.maximum, lanes)

                def _p2(g, acc):
                    base = g * (UNROLL * LANES)
                    for u in range(UNROLL):
                        x = buf[r, pl.ds(base + u * LANES, LANES)]
                        colv = (base + u * LANES + lanes) < m
                        e = jnp.where(colv, jnp.exp(x * ALPHA - rowmax), 0.0)
                        buf[r, pl.ds(base + u * LANES, LANES)] = e
                        acc = acc + e
                    return acc

                svec = lax.fori_loop(0, ngrp, _p2, zvec)
                denom = _all_lanes_reduce(svec, jnp.add, lanes)
                scale = 1.0 / denom

                def _p3(g, carry2):
                    base = g * (UNROLL * LANES)
                    for u in range(UNROLL):
                        buf[r, pl.ds(base + u * LANES, LANES)] = (
                            buf[r, pl.ds(base + u * LANES, LANES)] * scale)
                    return carry2

                lax.fori_loop(0, ngrp, _p3, 0)

                def _tail(g, carry2):
                    base = g * (UNROLL * LANES)
                    for u in range(UNROLL):
                        buf[r, pl.ds(base + u * LANES, LANES)] = zvec
                    return carry2

                lax.fori_loop(ngrp, CVECS // UNROLL, _tail, 0)
                return carry

            lax.fori_loop(0, rv, _row, 0)

            def _zrow(r, carry):
                def _z(g, carry2):
                    base = g * (UNROLL * LANES)
                    for u in range(UNROLL):
                        buf[r, pl.ds(base + u * LANES, LANES)] = zvec
                    return carry2

                lax.fori_loop(0, CVECS // UNROLL, _z, 0)
                return carry

            lax.fori_loop(rv, CHUNK, _zrow, 0)
            pltpu.sync_copy(buf, out_hbm.at[b, pl.ds(r0, CHUNK), :])

        @pl.when(r0 >= n)
        def _zero():
            pltpu.sync_copy(zbuf, out_hbm.at[b, pl.ds(r0, CHUNK), :])


@jax.jit
def _sm_call(s, nrow_gt, ncol_gt):
    mesh = plsc.VectorSubcoreMesh(core_axis_name="c", subcore_axis_name="s")
    return pl.kernel(
        _sm_body,
        mesh=mesh,
        out_type=jax.ShapeDtypeStruct((B, N, M), jnp.float32),
        scratch_types=[
            pltpu.VMEM((CHUNK, M), jnp.float32),
            pltpu.VMEM((CHUNK, M), jnp.float32),
            pltpu.VMEM((LANES,), jnp.int32),
            pltpu.VMEM((LANES,), jnp.int32),
        ],
    )(s, nrow_gt, ncol_gt)


def kernel(s, nrow_gt, ncol_gt):
    return _sm_call(s, nrow_gt, ncol_gt)
